# baseline (speedup 1.0000x reference)
import functools

import jax
import jax.numpy as jnp
from jax import lax
from jax.experimental import pallas as pl
from jax.experimental.pallas import tpu as pltpu
from jax.experimental.pallas import tpu_sc as plsc

N = 10000
E = 320000
D = 128
HID = 16
HEADS = 8
OUT = 64

NC = 2
NS = 16
NW = NC * NS
C = 128
CHUNKS = E // C
CPW = -(-CHUNKS // NW)
NP = N
RPT = NP // NS
ACC1 = 136
ACC2 = 80


def _proj1_body(x_ref, w_ref, aa_ref, h_ref, asad_ref):
    h = jnp.dot(x_ref[...], w_ref[...], preferred_element_type=jnp.float32)
    h_ref[...] = h
    asad_ref[...] = jnp.dot(h, aa_ref[...], preferred_element_type=jnp.float32)


def _combine1_body(accm_ref, accd_ref, r8_ref, b1_ref, w2_ref, a2_ref,
                   h2_ref, asad2_ref):
    num = accm_ref[0] + accm_ref[1]
    d = accd_ref[0] + accd_ref[1]
    den = d[:, :HEADS]
    den128 = jnp.dot(den, r8_ref[...], preferred_element_type=jnp.float32)
    h1 = num / (den128 + 1e-16) + b1_ref[...]
    act = jnp.where(h1 > 0, h1, jnp.exp(h1) - 1.0)
    h2 = jnp.dot(act, w2_ref[...], preferred_element_type=jnp.float32)
    h2_ref[...] = h2
    asad2_ref[...] = jnp.dot(h2, a2_ref[...], preferred_element_type=jnp.float32)


def _combine2_body(acc_ref, b2_ref, out_ref):
    a = acc_ref[0] + acc_ref[1]
    num = a[:, :OUT]
    den = a[:, OUT:OUT + 1]
    out_ref[...] = num / (den + 1e-16) + b2_ref[...]


_MESH = dict(core_axis_name="c", subcore_axis_name="s", num_cores=NC,
             num_subcores=NS)


def _zero_acc(buf, acc, s, width):
    nv = width // 16

    @plsc.parallel_loop(0, C * nv, 1, unroll=4)
    def _(r):
        buf[r // nv, pl.ds((r % nv) * 16, 16)] = jnp.zeros((16,), jnp.float32)

    nrows = 125
    for t in range(RPT // nrows):
        pltpu.sync_copy(buf.at[pl.ds(0, nrows)],
                        acc.at[pl.ds(s * RPT + t * nrows, nrows)])


def _edge1_kernel(sd, asad, h, outm, outd, srci, dsti, g1, g2, hrows, exb,
                  accm, accd, semg, semh):
    c = lax.axis_index("c")
    s = lax.axis_index("s")
    w = s * NC + c
    lanes = lax.iota(jnp.int32, 16)
    shift8 = lanes ^ 8
    hsel = [jnp.full((16,), hd, jnp.int32) for hd in range(HEADS)]

    _zero_acc(hrows, accm, s, D)
    _zero_acc(exb, accd, s, 16)
    plsc.subcore_barrier()

    def issue_idx_g(j, b):
        base = (w + NW * j) * C
        pltpu.sync_copy(sd.at[0, pl.ds(base, C)], srci)
        pltpu.sync_copy(sd.at[pl.ds(1, 1), pl.ds(base, C)], dsti.at[b])
        pltpu.async_copy(asad.at[srci], g1, semg)
        pltpu.async_copy(asad.at[dsti.at[b, 0]], g2, semg)

    def valid(j):
        return (w + NW * j) * C < E

    def body(j, b):
        pltpu.make_async_copy(asad.at[srci], g1, semg).wait()
        pltpu.make_async_copy(asad.at[dsti.at[b, 0]], g2, semg).wait()
        pltpu.async_copy(h.at[srci], hrows, semh)

        @plsc.parallel_loop(0, C, 1, unroll=4)
        def _(kk):
            v1 = g1[kk]
            v2 = g2[kk]
            e = v1 + jnp.take(v2, shift8)
            e = jnp.where(e > 0, e, 0.2 * e)
            e = jnp.where(lanes < 8, e, 0.0)
            exb[kk] = jnp.exp(e)

        pltpu.make_async_copy(h.at[srci], hrows, semh).wait()

        @pl.when(valid(j + 1))
        def _():
            issue_idx_g(j + 1, 1 - b)

        @plsc.parallel_loop(0, C, 1, unroll=2)
        def _(kk):
            ex = exb[kk]
            for hd in range(HEADS):
                wv = jnp.take(ex, hsel[hd])
                hrows[kk, pl.ds(hd * 16, 16)] = (
                    hrows[kk, pl.ds(hd * 16, 16)] * wv)

        pltpu.sync_copy(hrows, accm.at[dsti.at[b, 0]], add=True)
        pltpu.sync_copy(exb, accd.at[dsti.at[b, 0]], add=True)

    @pl.when(valid(0))
    def _():
        issue_idx_g(0, 0)

    def pair(i, _):
        j0 = 2 * i

        @pl.when(valid(j0))
        def _():
            body(j0, 0)

        @pl.when(valid(j0 + 1))
        def _():
            body(j0 + 1, 1)
        return 0

    lax.fori_loop(0, (CPW + 1) // 2, pair, 0)
    plsc.subcore_barrier()
    pltpu.sync_copy(accm.at[pl.ds(s * RPT, RPT)],
                    outm.at[c, pl.ds(s * RPT, RPT)])
    pltpu.sync_copy(accd.at[pl.ds(s * RPT, RPT)],
                    outd.at[c, pl.ds(s * RPT, RPT)])


def _edge2_kernel(sd, asad2, h2, out, srci, dsti, av, hrows, msg, acc,
                  sem0, sem1):
    sems = (sem0, sem1)
    c = lax.axis_index("c")
    s = lax.axis_index("s")
    w = s * NC + c
    lanes = lax.iota(jnp.int32, 16)
    jsel = [jnp.full((16,), j, jnp.int32) for j in range(16)]

    pltpu.sync_copy(asad2, av)
    _zero_acc(msg, acc, s, ACC2)
    plsc.subcore_barrier()

    def issue(j, b):
        base = (w + NW * j) * C
        pltpu.sync_copy(sd.at[0, pl.ds(base, C)], srci.at[b])
        pltpu.sync_copy(sd.at[pl.ds(1, 1), pl.ds(base, C)], dsti.at[b])
        pltpu.async_copy(h2.at[srci.at[b]], hrows.at[b], sems[b])

    def consume(b):
        pltpu.make_async_copy(h2.at[srci.at[b]], hrows.at[b], sems[b]).wait()

        @plsc.parallel_loop(0, C // 16, 1)
        def _(k):
            sv = plsc.load_gather(av.at[0], [srci[b, pl.ds(k * 16, 16)]])
            dv = plsc.load_gather(av.at[1], [dsti[b, 0, pl.ds(k * 16, 16)]])
            e = sv + dv
            e = jnp.where(e > 0, e, 0.2 * e)
            ex = jnp.exp(e)
            for j in range(16):
                kk = k * 16 + j
                wv = jnp.take(ex, jsel[j])
                for q in range(OUT // 16):
                    msg[kk, pl.ds(q * 16, 16)] = (
                        hrows[b, kk, pl.ds(q * 16, 16)] * wv)
                msg[kk, pl.ds(OUT, 16)] = jnp.where(lanes < 1, wv, 0.0)

        pltpu.sync_copy(msg, acc.at[dsti.at[b, 0]], add=True)

    def valid(j):
        return (w + NW * j) * C < E

    @pl.when(valid(0))
    def _():
        issue(0, 0)

    def pair(i, _):
        j0 = 2 * i

        @pl.when(valid(j0 + 1))
        def _():
            issue(j0 + 1, 1)

        @pl.when(valid(j0))
        def _():
            consume(0)

        @pl.when(valid(j0 + 2))
        def _():
            issue(j0 + 2, 0)

        @pl.when(valid(j0 + 1))
        def _():
            consume(1)
        return 0

    lax.fori_loop(0, (CPW + 1) // 2, pair, 0)
    plsc.subcore_barrier()
    pltpu.sync_copy(acc.at[pl.ds(s * RPT, RPT)], out.at[c, pl.ds(s * RPT, RPT)])


def kernel(x, edge_index, W1, a_src1, a_dst1, b1, W2, a_src2, a_dst2, b2):
    f32 = jnp.float32
    kk = jnp.arange(D)
    m1 = (kk[:, None] // HID == jnp.arange(HEADS)[None, :]).astype(f32)
    asad_w = jnp.concatenate([a_src1.reshape(-1)[:, None] * m1,
                              a_dst1.reshape(-1)[:, None] * m1], axis=1)
    r8 = (jnp.arange(HEADS)[:, None] == (jnp.arange(D)[None, :] // HID)
          ).astype(f32)
    a2 = jnp.concatenate([a_src2, a_dst2], axis=0).T

    BN = 2000
    grid = (N // BN,)

    h1, asad1 = pl.pallas_call(
        _proj1_body,
        grid=grid,
        in_specs=[
            pl.BlockSpec((BN, D), lambda i: (i, 0)),
            pl.BlockSpec((D, D), lambda i: (0, 0)),
            pl.BlockSpec((D, 2 * HEADS), lambda i: (0, 0)),
        ],
        out_specs=[
            pl.BlockSpec((BN, D), lambda i: (i, 0)),
            pl.BlockSpec((BN, 2 * HEADS), lambda i: (i, 0)),
        ],
        out_shape=[
            jax.ShapeDtypeStruct((N, D), f32),
            jax.ShapeDtypeStruct((N, 2 * HEADS), f32),
        ],
    )(x, W1, asad_w)

    mesh = plsc.VectorSubcoreMesh(**_MESH)

    edge1 = functools.partial(
        pl.kernel,
        out_type=(jax.ShapeDtypeStruct((NC, NP, D), f32),
                  jax.ShapeDtypeStruct((NC, NP, 16), f32)),
        mesh=mesh,
        compiler_params=pltpu.CompilerParams(use_tc_tiling_on_sc=False, needs_layout_passes=False),
        scratch_types=[
            pltpu.VMEM((C,), jnp.int32),
            pltpu.VMEM((2, 1, C), jnp.int32),
            pltpu.VMEM((C, 2 * HEADS), f32),
            pltpu.VMEM((C, 2 * HEADS), f32),
            pltpu.VMEM((C, D), f32),
            pltpu.VMEM((C, 16), f32),
            pltpu.VMEM_SHARED((NP, D), f32),
            pltpu.VMEM_SHARED((NP, 16), f32),
            pltpu.SemaphoreType.DMA,
            pltpu.SemaphoreType.DMA,
        ],
    )(_edge1_kernel)
    accm1, accd1 = edge1(edge_index, asad1, h1)

    h2, asad2 = pl.pallas_call(
        _combine1_body,
        grid=grid,
        in_specs=[
            pl.BlockSpec((NC, BN, D), lambda i: (0, i, 0)),
            pl.BlockSpec((NC, BN, 16), lambda i: (0, i, 0)),
            pl.BlockSpec((HEADS, D), lambda i: (0, 0)),
            pl.BlockSpec((1, D), lambda i: (0, 0)),
            pl.BlockSpec((D, OUT), lambda i: (0, 0)),
            pl.BlockSpec((OUT, 2), lambda i: (0, 0)),
        ],
        out_specs=[
            pl.BlockSpec((BN, OUT), lambda i: (i, 0)),
            pl.BlockSpec((BN, 2), lambda i: (i, 0)),
        ],
        out_shape=[
            jax.ShapeDtypeStruct((N, OUT), f32),
            jax.ShapeDtypeStruct((N, 2), f32),
        ],
    )(accm1, accd1, r8, b1.reshape(1, D), W2, a2)

    edge2 = functools.partial(
        pl.kernel,
        out_type=jax.ShapeDtypeStruct((NC, NP, ACC2), f32),
        mesh=mesh,
        compiler_params=pltpu.CompilerParams(use_tc_tiling_on_sc=False, needs_layout_passes=False),
        scratch_types=[
            pltpu.VMEM((2, C), jnp.int32),
            pltpu.VMEM((2, 1, C), jnp.int32),
            pltpu.VMEM((2, N), f32),
            pltpu.VMEM((2, C, OUT), f32),
            pltpu.VMEM((C, ACC2), f32),
            pltpu.VMEM_SHARED((NP, ACC2), f32),
            pltpu.SemaphoreType.DMA,
            pltpu.SemaphoreType.DMA,
        ],
    )(_edge2_kernel)
    acc2 = edge2(edge_index, asad2.T.reshape(2, N), h2)

    out = pl.pallas_call(
        _combine2_body,
        grid=grid,
        in_specs=[
            pl.BlockSpec((NC, BN, ACC2), lambda i: (0, i, 0)),
            pl.BlockSpec((1, OUT), lambda i: (0, 0)),
        ],
        out_specs=pl.BlockSpec((BN, OUT), lambda i: (i, 0)),
        out_shape=jax.ShapeDtypeStruct((N, OUT), f32),
    )(acc2, b2.reshape(1, OUT))
    return out


# speedup vs baseline: 1.0023x; 1.0023x over previous
import functools

import jax
import jax.numpy as jnp
from jax import lax
from jax.experimental import pallas as pl
from jax.experimental.pallas import tpu as pltpu
from jax.experimental.pallas import tpu_sc as plsc

N = 10000
E = 320000
D = 128
HID = 16
HEADS = 8
OUT = 64

NC = 2
NS = 16
NW = NC * NS
C = 128
CHUNKS = E // C
CPW = -(-CHUNKS // NW)
NP = N
RPT = NP // NS
ACC1 = 136
ACC2 = 80


def _proj1_body(x_ref, w_ref, aa_ref, h_ref, asad_ref):
    h = jnp.dot(x_ref[...], w_ref[...], preferred_element_type=jnp.float32)
    h_ref[...] = h
    asad_ref[...] = jnp.dot(h, aa_ref[...], preferred_element_type=jnp.float32)


def _combine1_body(accm_ref, accd_ref, r8_ref, b1_ref, w2_ref, a2_ref,
                   h2_ref, asad2_ref):
    num = accm_ref[0] + accm_ref[1]
    d = accd_ref[0] + accd_ref[1]
    den = d[:, :HEADS]
    den128 = jnp.dot(den, r8_ref[...], preferred_element_type=jnp.float32)
    h1 = num / (den128 + 1e-16) + b1_ref[...]
    act = jnp.where(h1 > 0, h1, jnp.exp(h1) - 1.0)
    h2 = jnp.dot(act, w2_ref[...], preferred_element_type=jnp.float32)
    h2_ref[...] = h2
    asad2_ref[...] = jnp.dot(h2, a2_ref[...], preferred_element_type=jnp.float32)


def _combine2_body(acc_ref, b2_ref, out_ref):
    a = acc_ref[0] + acc_ref[1]
    num = a[:, :OUT]
    den = a[:, OUT:OUT + 1]
    out_ref[...] = num / (den + 1e-16) + b2_ref[...]


_MESH = dict(core_axis_name="c", subcore_axis_name="s", num_cores=NC,
             num_subcores=NS)


def _zero_acc(buf, acc, s, width):
    nv = width // 16

    @plsc.parallel_loop(0, C * nv, 1, unroll=4)
    def _(r):
        buf[r // nv, pl.ds((r % nv) * 16, 16)] = jnp.zeros((16,), jnp.float32)

    nrows = 125
    for t in range(RPT // nrows):
        pltpu.sync_copy(buf.at[pl.ds(0, nrows)],
                        acc.at[pl.ds(s * RPT + t * nrows, nrows)])


def _edge1_kernel(sd, asad, h, outm, outd, srci, dsti, g1, g2, hrows, exb,
                  accm, accd, semg, semh):
    c = lax.axis_index("c")
    s = lax.axis_index("s")
    w = s * NC + c
    lanes = lax.iota(jnp.int32, 16)
    shift8 = lanes ^ 8
    hsel = [jnp.full((16,), hd, jnp.int32) for hd in range(HEADS)]

    _zero_acc(hrows, accm, s, D)
    _zero_acc(exb, accd, s, 16)
    plsc.subcore_barrier()

    def issue_idx_g(j, b):
        base = (w + NW * j) * C
        pltpu.sync_copy(sd.at[0, pl.ds(base, C)], srci)
        pltpu.sync_copy(sd.at[pl.ds(1, 1), pl.ds(base, C)], dsti.at[b])
        pltpu.async_copy(asad.at[srci], g1, semg)
        pltpu.async_copy(asad.at[dsti.at[b, 0]], g2, semg)

    def valid(j):
        return (w + NW * j) * C < E

    def body(j, b):
        pltpu.make_async_copy(asad.at[srci], g1, semg).wait()
        pltpu.make_async_copy(asad.at[dsti.at[b, 0]], g2, semg).wait()
        pltpu.async_copy(h.at[srci], hrows, semh)

        @plsc.parallel_loop(0, C, 1, unroll=4)
        def _(kk):
            v1 = g1[kk]
            v2 = g2[kk]
            e = v1 + jnp.take(v2, shift8)
            e = jnp.where(e > 0, e, 0.2 * e)
            e = jnp.where(lanes < 8, e, 0.0)
            exb[kk] = jnp.exp(e)

        pltpu.make_async_copy(h.at[srci], hrows, semh).wait()

        @pl.when(valid(j + 1))
        def _():
            issue_idx_g(j + 1, 1 - b)

        @plsc.parallel_loop(0, C, 1, unroll=4)
        def _(kk):
            ex = exb[kk]
            for hd in range(HEADS):
                wv = jnp.take(ex, hsel[hd])
                hrows[kk, pl.ds(hd * 16, 16)] = (
                    hrows[kk, pl.ds(hd * 16, 16)] * wv)

        pltpu.sync_copy(hrows, accm.at[dsti.at[b, 0]], add=True)
        pltpu.sync_copy(exb, accd.at[dsti.at[b, 0]], add=True)

    @pl.when(valid(0))
    def _():
        issue_idx_g(0, 0)

    def pair(i, _):
        j0 = 2 * i

        @pl.when(valid(j0))
        def _():
            body(j0, 0)

        @pl.when(valid(j0 + 1))
        def _():
            body(j0 + 1, 1)
        return 0

    lax.fori_loop(0, (CPW + 1) // 2, pair, 0)
    plsc.subcore_barrier()
    pltpu.sync_copy(accm.at[pl.ds(s * RPT, RPT)],
                    outm.at[c, pl.ds(s * RPT, RPT)])
    pltpu.sync_copy(accd.at[pl.ds(s * RPT, RPT)],
                    outd.at[c, pl.ds(s * RPT, RPT)])


def _edge2_kernel(sd, asad2, h2, out, srci, dsti, av, hrows, msg, acc,
                  sem0, sem1):
    sems = (sem0, sem1)
    c = lax.axis_index("c")
    s = lax.axis_index("s")
    w = s * NC + c
    lanes = lax.iota(jnp.int32, 16)
    jsel = [jnp.full((16,), j, jnp.int32) for j in range(16)]

    pltpu.sync_copy(asad2, av)
    _zero_acc(msg, acc, s, ACC2)
    plsc.subcore_barrier()

    def issue(j, b):
        base = (w + NW * j) * C
        pltpu.sync_copy(sd.at[0, pl.ds(base, C)], srci.at[b])
        pltpu.sync_copy(sd.at[pl.ds(1, 1), pl.ds(base, C)], dsti.at[b])
        pltpu.async_copy(h2.at[srci.at[b]], hrows.at[b], sems[b])

    def consume(b):
        pltpu.make_async_copy(h2.at[srci.at[b]], hrows.at[b], sems[b]).wait()

        @plsc.parallel_loop(0, C // 16, 1)
        def _(k):
            sv = plsc.load_gather(av.at[0], [srci[b, pl.ds(k * 16, 16)]])
            dv = plsc.load_gather(av.at[1], [dsti[b, 0, pl.ds(k * 16, 16)]])
            e = sv + dv
            e = jnp.where(e > 0, e, 0.2 * e)
            ex = jnp.exp(e)
            for j in range(16):
                kk = k * 16 + j
                wv = jnp.take(ex, jsel[j])
                for q in range(OUT // 16):
                    msg[kk, pl.ds(q * 16, 16)] = (
                        hrows[b, kk, pl.ds(q * 16, 16)] * wv)
                msg[kk, pl.ds(OUT, 16)] = jnp.where(lanes < 1, wv, 0.0)

        pltpu.sync_copy(msg, acc.at[dsti.at[b, 0]], add=True)

    def valid(j):
        return (w + NW * j) * C < E

    @pl.when(valid(0))
    def _():
        issue(0, 0)

    def pair(i, _):
        j0 = 2 * i

        @pl.when(valid(j0 + 1))
        def _():
            issue(j0 + 1, 1)

        @pl.when(valid(j0))
        def _():
            consume(0)

        @pl.when(valid(j0 + 2))
        def _():
            issue(j0 + 2, 0)

        @pl.when(valid(j0 + 1))
        def _():
            consume(1)
        return 0

    lax.fori_loop(0, (CPW + 1) // 2, pair, 0)
    plsc.subcore_barrier()
    pltpu.sync_copy(acc.at[pl.ds(s * RPT, RPT)], out.at[c, pl.ds(s * RPT, RPT)])


def kernel(x, edge_index, W1, a_src1, a_dst1, b1, W2, a_src2, a_dst2, b2):
    f32 = jnp.float32
    kk = jnp.arange(D)
    m1 = (kk[:, None] // HID == jnp.arange(HEADS)[None, :]).astype(f32)
    asad_w = jnp.concatenate([a_src1.reshape(-1)[:, None] * m1,
                              a_dst1.reshape(-1)[:, None] * m1], axis=1)
    r8 = (jnp.arange(HEADS)[:, None] == (jnp.arange(D)[None, :] // HID)
          ).astype(f32)
    a2 = jnp.concatenate([a_src2, a_dst2], axis=0).T

    BN = 2000
    grid = (N // BN,)

    h1, asad1 = pl.pallas_call(
        _proj1_body,
        grid=grid,
        in_specs=[
            pl.BlockSpec((BN, D), lambda i: (i, 0)),
            pl.BlockSpec((D, D), lambda i: (0, 0)),
            pl.BlockSpec((D, 2 * HEADS), lambda i: (0, 0)),
        ],
        out_specs=[
            pl.BlockSpec((BN, D), lambda i: (i, 0)),
            pl.BlockSpec((BN, 2 * HEADS), lambda i: (i, 0)),
        ],
        out_shape=[
            jax.ShapeDtypeStruct((N, D), f32),
            jax.ShapeDtypeStruct((N, 2 * HEADS), f32),
        ],
    )(x, W1, asad_w)

    mesh = plsc.VectorSubcoreMesh(**_MESH)

    edge1 = functools.partial(
        pl.kernel,
        out_type=(jax.ShapeDtypeStruct((NC, NP, D), f32),
                  jax.ShapeDtypeStruct((NC, NP, 16), f32)),
        mesh=mesh,
        compiler_params=pltpu.CompilerParams(use_tc_tiling_on_sc=False, needs_layout_passes=False),
        scratch_types=[
            pltpu.VMEM((C,), jnp.int32),
            pltpu.VMEM((2, 1, C), jnp.int32),
            pltpu.VMEM((C, 2 * HEADS), f32),
            pltpu.VMEM((C, 2 * HEADS), f32),
            pltpu.VMEM((C, D), f32),
            pltpu.VMEM((C, 16), f32),
            pltpu.VMEM_SHARED((NP, D), f32),
            pltpu.VMEM_SHARED((NP, 16), f32),
            pltpu.SemaphoreType.DMA,
            pltpu.SemaphoreType.DMA,
        ],
    )(_edge1_kernel)
    accm1, accd1 = edge1(edge_index, asad1, h1)

    h2, asad2 = pl.pallas_call(
        _combine1_body,
        grid=grid,
        in_specs=[
            pl.BlockSpec((NC, BN, D), lambda i: (0, i, 0)),
            pl.BlockSpec((NC, BN, 16), lambda i: (0, i, 0)),
            pl.BlockSpec((HEADS, D), lambda i: (0, 0)),
            pl.BlockSpec((1, D), lambda i: (0, 0)),
            pl.BlockSpec((D, OUT), lambda i: (0, 0)),
            pl.BlockSpec((OUT, 2), lambda i: (0, 0)),
        ],
        out_specs=[
            pl.BlockSpec((BN, OUT), lambda i: (i, 0)),
            pl.BlockSpec((BN, 2), lambda i: (i, 0)),
        ],
        out_shape=[
            jax.ShapeDtypeStruct((N, OUT), f32),
            jax.ShapeDtypeStruct((N, 2), f32),
        ],
    )(accm1, accd1, r8, b1.reshape(1, D), W2, a2)

    edge2 = functools.partial(
        pl.kernel,
        out_type=jax.ShapeDtypeStruct((NC, NP, ACC2), f32),
        mesh=mesh,
        compiler_params=pltpu.CompilerParams(use_tc_tiling_on_sc=False, needs_layout_passes=False),
        scratch_types=[
            pltpu.VMEM((2, C), jnp.int32),
            pltpu.VMEM((2, 1, C), jnp.int32),
            pltpu.VMEM((2, N), f32),
            pltpu.VMEM((2, C, OUT), f32),
            pltpu.VMEM((C, ACC2), f32),
            pltpu.VMEM_SHARED((NP, ACC2), f32),
            pltpu.SemaphoreType.DMA,
            pltpu.SemaphoreType.DMA,
        ],
    )(_edge2_kernel)
    acc2 = edge2(edge_index, asad2.T.reshape(2, N), h2)

    out = pl.pallas_call(
        _combine2_body,
        grid=grid,
        in_specs=[
            pl.BlockSpec((NC, BN, ACC2), lambda i: (0, i, 0)),
            pl.BlockSpec((1, OUT), lambda i: (0, 0)),
        ],
        out_specs=pl.BlockSpec((BN, OUT), lambda i: (i, 0)),
        out_shape=jax.ShapeDtypeStruct((N, OUT), f32),
    )(acc2, b2.reshape(1, OUT))
    return out


# speedup vs baseline: 1.0710x; 1.0685x over previous
import functools

import jax
import jax.numpy as jnp
from jax import lax
from jax.experimental import pallas as pl
from jax.experimental.pallas import tpu as pltpu
from jax.experimental.pallas import tpu_sc as plsc

N = 10000
E = 320000
D = 128
HID = 16
HEADS = 8
OUT = 64

NC = 2
NS = 16
NW = NC * NS
C = 128
CHUNKS = E // C
CPW = -(-CHUNKS // NW)
NP = N
RPT = NP // NS
ACC1 = 136
ACC2 = 80


def _proj1_body(x_ref, w_ref, aa_ref, h_ref, asad_ref):
    h = jnp.dot(x_ref[...], w_ref[...], preferred_element_type=jnp.float32)
    h_ref[...] = h
    asad_ref[...] = jnp.dot(h, aa_ref[...], preferred_element_type=jnp.float32)


def _combine1_body(accm_ref, accd_ref, r8_ref, b1_ref, w2_ref, a2_ref,
                   h2_ref, asad2_ref):
    num = accm_ref[0] + accm_ref[1]
    d = accd_ref[0] + accd_ref[1]
    den = d[:, :HEADS]
    den128 = jnp.dot(den, r8_ref[...], preferred_element_type=jnp.float32)
    h1 = num / (den128 + 1e-16) + b1_ref[...]
    act = jnp.where(h1 > 0, h1, jnp.exp(h1) - 1.0)
    h2 = jnp.dot(act, w2_ref[...], preferred_element_type=jnp.float32)
    h2_ref[...] = h2
    asad2_ref[...] = jnp.dot(h2, a2_ref[...], preferred_element_type=jnp.float32)


def _combine2_body(acc_ref, b2_ref, out_ref):
    a = acc_ref[0] + acc_ref[1]
    num = a[:, :OUT]
    den = a[:, OUT:OUT + 1]
    out_ref[...] = num / (den + 1e-16) + b2_ref[...]


_MESH = dict(core_axis_name="c", subcore_axis_name="s", num_cores=NC,
             num_subcores=NS)


def _zero_acc(buf, acc, s, width):
    nv = width // 16

    @plsc.parallel_loop(0, C * nv, 1, unroll=4)
    def _(r):
        buf[r // nv, pl.ds((r % nv) * 16, 16)] = jnp.zeros((16,), jnp.float32)

    nrows = 125
    for t in range(RPT // nrows):
        pltpu.sync_copy(buf.at[pl.ds(0, nrows)],
                        acc.at[pl.ds(s * RPT + t * nrows, nrows)])


def _edge1_kernel(sd, asad, h, outm, outd, srci, dsti, g1, g2, hrows, exb,
                  accm, accd, semg, semh, semsc0, semsc1):
    semsc = (semsc0, semsc1)
    c = lax.axis_index("c")
    s = lax.axis_index("s")
    w = s * NC + c
    lanes = lax.iota(jnp.int32, 16)
    shift8 = lanes ^ 8
    hsel = [jnp.full((16,), hd, jnp.int32) for hd in range(HEADS)]

    _zero_acc(hrows.at[0], accm, s, D)
    _zero_acc(exb, accd, s, 16)
    plsc.subcore_barrier()

    def issue_idx_g(j, b):
        base = (w + NW * j) * C
        pltpu.sync_copy(sd.at[0, pl.ds(base, C)], srci)
        pltpu.sync_copy(sd.at[pl.ds(1, 1), pl.ds(base, C)], dsti.at[b])
        pltpu.async_copy(asad.at[srci], g1, semg)
        pltpu.async_copy(asad.at[dsti.at[b, 0]], g2, semg)

    def valid(j):
        return (w + NW * j) * C < E

    def wait_scatter(b):
        pltpu.make_async_copy(hrows.at[b], accm.at[dsti.at[b, 0]],
                              semsc[b]).wait()

    def body(j, b):
        pltpu.make_async_copy(asad.at[srci], g1, semg).wait()
        pltpu.make_async_copy(asad.at[dsti.at[b, 0]], g2, semg).wait()
        pltpu.async_copy(h.at[srci], hrows.at[b], semh)

        @plsc.parallel_loop(0, C, 1, unroll=4)
        def _(kk):
            v1 = g1[kk]
            v2 = g2[kk]
            e = v1 + jnp.take(v2, shift8)
            e = jnp.where(e > 0, e, 0.2 * e)
            e = jnp.where(lanes < 8, e, 0.0)
            exb[kk] = jnp.exp(e)

        pltpu.make_async_copy(h.at[srci], hrows.at[b], semh).wait()

        @pl.when(jnp.logical_and(j >= 1, valid(j + 1)))
        def _():
            wait_scatter(1 - b)

        @pl.when(valid(j + 1))
        def _():
            issue_idx_g(j + 1, 1 - b)

        @plsc.parallel_loop(0, C, 1, unroll=4)
        def _(kk):
            ex = exb[kk]
            for hd in range(HEADS):
                wv = jnp.take(ex, hsel[hd])
                hrows[b, kk, pl.ds(hd * 16, 16)] = (
                    hrows[b, kk, pl.ds(hd * 16, 16)] * wv)

        pltpu.async_copy(hrows.at[b], accm.at[dsti.at[b, 0]], semsc[b],
                         add=True)
        pltpu.sync_copy(exb, accd.at[dsti.at[b, 0]], add=True)

    @pl.when(valid(0))
    def _():
        issue_idx_g(0, 0)

    def pair(i, _):
        j0 = 2 * i

        @pl.when(valid(j0))
        def _():
            body(j0, 0)

        @pl.when(valid(j0 + 1))
        def _():
            body(j0 + 1, 1)
        return 0

    lax.fori_loop(0, (CPW + 1) // 2, pair, 0)
    wait_scatter(0)
    wait_scatter(1)
    plsc.subcore_barrier()
    pltpu.sync_copy(accm.at[pl.ds(s * RPT, RPT)],
                    outm.at[c, pl.ds(s * RPT, RPT)])
    pltpu.sync_copy(accd.at[pl.ds(s * RPT, RPT)],
                    outd.at[c, pl.ds(s * RPT, RPT)])


def _edge2_kernel(sd, asad2, h2, out, srci, dsti, av, hrows, msg, acc,
                  sem0, sem1):
    sems = (sem0, sem1)
    c = lax.axis_index("c")
    s = lax.axis_index("s")
    w = s * NC + c
    lanes = lax.iota(jnp.int32, 16)
    jsel = [jnp.full((16,), j, jnp.int32) for j in range(16)]

    pltpu.sync_copy(asad2, av)
    _zero_acc(msg, acc, s, ACC2)
    plsc.subcore_barrier()

    def issue(j, b):
        base = (w + NW * j) * C
        pltpu.sync_copy(sd.at[0, pl.ds(base, C)], srci.at[b])
        pltpu.sync_copy(sd.at[pl.ds(1, 1), pl.ds(base, C)], dsti.at[b])
        pltpu.async_copy(h2.at[srci.at[b]], hrows.at[b], sems[b])

    def consume(b):
        pltpu.make_async_copy(h2.at[srci.at[b]], hrows.at[b], sems[b]).wait()

        @plsc.parallel_loop(0, C // 16, 1)
        def _(k):
            sv = plsc.load_gather(av.at[0], [srci[b, pl.ds(k * 16, 16)]])
            dv = plsc.load_gather(av.at[1], [dsti[b, 0, pl.ds(k * 16, 16)]])
            e = sv + dv
            e = jnp.where(e > 0, e, 0.2 * e)
            ex = jnp.exp(e)
            for j in range(16):
                kk = k * 16 + j
                wv = jnp.take(ex, jsel[j])
                for q in range(OUT // 16):
                    msg[kk, pl.ds(q * 16, 16)] = (
                        hrows[b, kk, pl.ds(q * 16, 16)] * wv)
                msg[kk, pl.ds(OUT, 16)] = jnp.where(lanes < 1, wv, 0.0)

        pltpu.sync_copy(msg, acc.at[dsti.at[b, 0]], add=True)

    def valid(j):
        return (w + NW * j) * C < E

    @pl.when(valid(0))
    def _():
        issue(0, 0)

    def pair(i, _):
        j0 = 2 * i

        @pl.when(valid(j0 + 1))
        def _():
            issue(j0 + 1, 1)

        @pl.when(valid(j0))
        def _():
            consume(0)

        @pl.when(valid(j0 + 2))
        def _():
            issue(j0 + 2, 0)

        @pl.when(valid(j0 + 1))
        def _():
            consume(1)
        return 0

    lax.fori_loop(0, (CPW + 1) // 2, pair, 0)
    plsc.subcore_barrier()
    pltpu.sync_copy(acc.at[pl.ds(s * RPT, RPT)], out.at[c, pl.ds(s * RPT, RPT)])


def kernel(x, edge_index, W1, a_src1, a_dst1, b1, W2, a_src2, a_dst2, b2):
    f32 = jnp.float32
    kk = jnp.arange(D)
    m1 = (kk[:, None] // HID == jnp.arange(HEADS)[None, :]).astype(f32)
    asad_w = jnp.concatenate([a_src1.reshape(-1)[:, None] * m1,
                              a_dst1.reshape(-1)[:, None] * m1], axis=1)
    r8 = (jnp.arange(HEADS)[:, None] == (jnp.arange(D)[None, :] // HID)
          ).astype(f32)
    a2 = jnp.concatenate([a_src2, a_dst2], axis=0).T

    BN = 2000
    grid = (N // BN,)

    h1, asad1 = pl.pallas_call(
        _proj1_body,
        grid=grid,
        in_specs=[
            pl.BlockSpec((BN, D), lambda i: (i, 0)),
            pl.BlockSpec((D, D), lambda i: (0, 0)),
            pl.BlockSpec((D, 2 * HEADS), lambda i: (0, 0)),
        ],
        out_specs=[
            pl.BlockSpec((BN, D), lambda i: (i, 0)),
            pl.BlockSpec((BN, 2 * HEADS), lambda i: (i, 0)),
        ],
        out_shape=[
            jax.ShapeDtypeStruct((N, D), f32),
            jax.ShapeDtypeStruct((N, 2 * HEADS), f32),
        ],
    )(x, W1, asad_w)

    mesh = plsc.VectorSubcoreMesh(**_MESH)

    edge1 = functools.partial(
        pl.kernel,
        out_type=(jax.ShapeDtypeStruct((NC, NP, D), f32),
                  jax.ShapeDtypeStruct((NC, NP, 16), f32)),
        mesh=mesh,
        compiler_params=pltpu.CompilerParams(use_tc_tiling_on_sc=False, needs_layout_passes=False),
        scratch_types=[
            pltpu.VMEM((C,), jnp.int32),
            pltpu.VMEM((2, 1, C), jnp.int32),
            pltpu.VMEM((C, 2 * HEADS), f32),
            pltpu.VMEM((C, 2 * HEADS), f32),
            pltpu.VMEM((2, C, D), f32),
            pltpu.VMEM((C, 16), f32),
            pltpu.VMEM_SHARED((NP, D), f32),
            pltpu.VMEM_SHARED((NP, 16), f32),
            pltpu.SemaphoreType.DMA,
            pltpu.SemaphoreType.DMA,
            pltpu.SemaphoreType.DMA,
            pltpu.SemaphoreType.DMA,
        ],
    )(_edge1_kernel)
    accm1, accd1 = edge1(edge_index, asad1, h1)

    h2, asad2 = pl.pallas_call(
        _combine1_body,
        grid=grid,
        in_specs=[
            pl.BlockSpec((NC, BN, D), lambda i: (0, i, 0)),
            pl.BlockSpec((NC, BN, 16), lambda i: (0, i, 0)),
            pl.BlockSpec((HEADS, D), lambda i: (0, 0)),
            pl.BlockSpec((1, D), lambda i: (0, 0)),
            pl.BlockSpec((D, OUT), lambda i: (0, 0)),
            pl.BlockSpec((OUT, 2), lambda i: (0, 0)),
        ],
        out_specs=[
            pl.BlockSpec((BN, OUT), lambda i: (i, 0)),
            pl.BlockSpec((BN, 2), lambda i: (i, 0)),
        ],
        out_shape=[
            jax.ShapeDtypeStruct((N, OUT), f32),
            jax.ShapeDtypeStruct((N, 2), f32),
        ],
    )(accm1, accd1, r8, b1.reshape(1, D), W2, a2)

    edge2 = functools.partial(
        pl.kernel,
        out_type=jax.ShapeDtypeStruct((NC, NP, ACC2), f32),
        mesh=mesh,
        compiler_params=pltpu.CompilerParams(use_tc_tiling_on_sc=False, needs_layout_passes=False),
        scratch_types=[
            pltpu.VMEM((2, C), jnp.int32),
            pltpu.VMEM((2, 1, C), jnp.int32),
            pltpu.VMEM((2, N), f32),
            pltpu.VMEM((2, C, OUT), f32),
            pltpu.VMEM((C, ACC2), f32),
            pltpu.VMEM_SHARED((NP, ACC2), f32),
            pltpu.SemaphoreType.DMA,
            pltpu.SemaphoreType.DMA,
        ],
    )(_edge2_kernel)
    acc2 = edge2(edge_index, asad2.T.reshape(2, N), h2)

    out = pl.pallas_call(
        _combine2_body,
        grid=grid,
        in_specs=[
            pl.BlockSpec((NC, BN, ACC2), lambda i: (0, i, 0)),
            pl.BlockSpec((1, OUT), lambda i: (0, 0)),
        ],
        out_specs=pl.BlockSpec((BN, OUT), lambda i: (i, 0)),
        out_shape=jax.ShapeDtypeStruct((N, OUT), f32),
    )(acc2, b2.reshape(1, OUT))
    return out


# speedup vs baseline: 1.1063x; 1.0330x over previous
import functools

import jax
import jax.numpy as jnp
from jax import lax
from jax.experimental import pallas as pl
from jax.experimental.pallas import tpu as pltpu
from jax.experimental.pallas import tpu_sc as plsc

N = 10000
E = 320000
D = 128
HID = 16
HEADS = 8
OUT = 64

NC = 2
NS = 16
NW = NC * NS
C = 128
CHUNKS = E // C
CPW = -(-CHUNKS // NW)
NP = N
RPT = NP // NS
ACC1 = 136
ACC2 = 80


def _proj1_body(x_ref, w_ref, aa_ref, h_ref, asad_ref):
    h = jnp.dot(x_ref[...], w_ref[...], preferred_element_type=jnp.float32)
    h_ref[...] = h
    asad_ref[...] = jnp.dot(h, aa_ref[...], preferred_element_type=jnp.float32)


def _combine1_body(accm_ref, accd_ref, r8_ref, b1_ref, w2_ref, a2_ref,
                   h2_ref, asad2_ref):
    num = accm_ref[0] + accm_ref[1]
    d = accd_ref[0] + accd_ref[1]
    den = d[:, :HEADS]
    den128 = jnp.dot(den, r8_ref[...], preferred_element_type=jnp.float32)
    h1 = num / (den128 + 1e-16) + b1_ref[...]
    act = jnp.where(h1 > 0, h1, jnp.exp(h1) - 1.0)
    h2 = jnp.dot(act, w2_ref[...], preferred_element_type=jnp.float32)
    h2_ref[...] = h2
    asad2_ref[...] = jnp.dot(h2, a2_ref[...], preferred_element_type=jnp.float32)


def _combine2_body(acc_ref, b2_ref, out_ref):
    a = acc_ref[0] + acc_ref[1]
    num = a[:, :OUT]
    den = a[:, OUT:OUT + 1]
    out_ref[...] = num / (den + 1e-16) + b2_ref[...]


_MESH = dict(core_axis_name="c", subcore_axis_name="s", num_cores=NC,
             num_subcores=NS)


def _zero_acc(buf, acc, s, width):
    nv = width // 16

    @plsc.parallel_loop(0, C * nv, 1, unroll=4)
    def _(r):
        buf[r // nv, pl.ds((r % nv) * 16, 16)] = jnp.zeros((16,), jnp.float32)

    nrows = 125
    for t in range(RPT // nrows):
        pltpu.sync_copy(buf.at[pl.ds(0, nrows)],
                        acc.at[pl.ds(s * RPT + t * nrows, nrows)])


def _edge1_kernel(sd, asad, h, outm, outd, srci, dsti, g1, g2, hrows, exb,
                  accm, accd, semg, semh, semsc0, semsc1):
    semsc = (semsc0, semsc1)
    c = lax.axis_index("c")
    s = lax.axis_index("s")
    w = s * NC + c
    lanes = lax.iota(jnp.int32, 16)
    shift8 = lanes ^ 8
    hsel = [jnp.full((16,), hd, jnp.int32) for hd in range(HEADS)]

    _zero_acc(hrows.at[0], accm, s, D)
    _zero_acc(exb, accd, s, 16)
    plsc.subcore_barrier()

    def issue_idx_g(j, b):
        base = (w + NW * j) * C
        pltpu.sync_copy(sd.at[0, pl.ds(base, C)], srci)
        pltpu.sync_copy(sd.at[pl.ds(1, 1), pl.ds(base, C)], dsti.at[b])
        pltpu.async_copy(asad.at[srci], g1, semg)
        pltpu.async_copy(asad.at[dsti.at[b, 0]], g2, semg)

    def valid(j):
        return (w + NW * j) * C < E

    def wait_scatter(b):
        pltpu.make_async_copy(hrows.at[b], accm.at[dsti.at[b, 0]],
                              semsc[b]).wait()

    def body(j, b):
        pltpu.make_async_copy(asad.at[srci], g1, semg).wait()
        pltpu.make_async_copy(asad.at[dsti.at[b, 0]], g2, semg).wait()
        pltpu.async_copy(h.at[srci], hrows.at[b], semh)

        @plsc.parallel_loop(0, C, 1, unroll=4)
        def _(kk):
            v1 = g1[kk]
            v2 = g2[kk]
            e = v1 + jnp.take(v2, shift8)
            e = jnp.where(e > 0, e, 0.2 * e)
            e = jnp.where(lanes < 8, e, 0.0)
            exb[kk] = jnp.exp(e)

        pltpu.make_async_copy(h.at[srci], hrows.at[b], semh).wait()

        @pl.when(jnp.logical_and(j >= 1, valid(j + 1)))
        def _():
            wait_scatter(1 - b)

        @pl.when(valid(j + 1))
        def _():
            issue_idx_g(j + 1, 1 - b)

        @plsc.parallel_loop(0, C, 1, unroll=4)
        def _(kk):
            ex = exb[kk]
            for hd in range(HEADS):
                wv = jnp.take(ex, hsel[hd])
                hrows[b, kk, pl.ds(hd * 16, 16)] = (
                    hrows[b, kk, pl.ds(hd * 16, 16)] * wv)

        pltpu.async_copy(hrows.at[b], accm.at[dsti.at[b, 0]], semsc[b],
                         add=True)
        pltpu.sync_copy(exb, accd.at[dsti.at[b, 0]], add=True)

    @pl.when(valid(0))
    def _():
        issue_idx_g(0, 0)

    def pair(i, _):
        j0 = 2 * i

        @pl.when(valid(j0))
        def _():
            body(j0, 0)

        @pl.when(valid(j0 + 1))
        def _():
            body(j0 + 1, 1)
        return 0

    lax.fori_loop(0, (CPW + 1) // 2, pair, 0)
    wait_scatter(0)
    wait_scatter(1)
    plsc.subcore_barrier()
    pltpu.sync_copy(accm.at[pl.ds(s * RPT, RPT)],
                    outm.at[c, pl.ds(s * RPT, RPT)])
    pltpu.sync_copy(accd.at[pl.ds(s * RPT, RPT)],
                    outd.at[c, pl.ds(s * RPT, RPT)])


def _edge2_kernel(sd, asad2, h2, out, srci, dsti, dscat, av, hrows, msg, acc,
                  sem0, sem1, semsc0, semsc1):
    sems = (sem0, sem1)
    semsc = (semsc0, semsc1)
    c = lax.axis_index("c")
    s = lax.axis_index("s")
    w = s * NC + c
    lanes = lax.iota(jnp.int32, 16)
    jsel = [jnp.full((16,), j, jnp.int32) for j in range(16)]

    pltpu.sync_copy(asad2, av)
    _zero_acc(msg.at[0], acc, s, ACC2)
    plsc.subcore_barrier()

    def issue(j, b):
        base = (w + NW * j) * C
        pltpu.sync_copy(sd.at[0, pl.ds(base, C)], srci.at[b])
        pltpu.sync_copy(sd.at[pl.ds(1, 1), pl.ds(base, C)], dsti.at[b])
        pltpu.async_copy(h2.at[srci.at[b]], hrows.at[b], sems[b])

    def wait_scatter(b):
        pltpu.make_async_copy(msg.at[b], acc.at[dscat.at[b, 0]],
                              semsc[b]).wait()

    def consume(j, b):
        pltpu.make_async_copy(h2.at[srci.at[b]], hrows.at[b], sems[b]).wait()

        @pl.when(j >= 2)
        def _():
            wait_scatter(b)

        @plsc.parallel_loop(0, C // 16, 1)
        def _(k):
            sv = plsc.load_gather(av.at[0], [srci[b, pl.ds(k * 16, 16)]])
            dv = plsc.load_gather(av.at[1], [dsti[b, 0, pl.ds(k * 16, 16)]])
            e = sv + dv
            e = jnp.where(e > 0, e, 0.2 * e)
            ex = jnp.exp(e)
            for j2 in range(16):
                kk = k * 16 + j2
                wv = jnp.take(ex, jsel[j2])
                for q in range(OUT // 16):
                    msg[b, kk, pl.ds(q * 16, 16)] = (
                        hrows[b, kk, pl.ds(q * 16, 16)] * wv)
                msg[b, kk, pl.ds(OUT, 16)] = jnp.where(lanes < 1, wv, 0.0)

        @plsc.parallel_loop(0, C // 16, 1)
        def _(k):
            dscat[b, 0, pl.ds(k * 16, 16)] = dsti[b, 0, pl.ds(k * 16, 16)]
        pltpu.async_copy(msg.at[b], acc.at[dscat.at[b, 0]], semsc[b],
                         add=True)

    def valid(j):
        return (w + NW * j) * C < E

    @pl.when(valid(0))
    def _():
        issue(0, 0)

    def pair(i, _):
        j0 = 2 * i

        @pl.when(valid(j0 + 1))
        def _():
            issue(j0 + 1, 1)

        @pl.when(valid(j0))
        def _():
            consume(j0, 0)

        @pl.when(valid(j0 + 2))
        def _():
            issue(j0 + 2, 0)

        @pl.when(valid(j0 + 1))
        def _():
            consume(j0 + 1, 1)
        return 0

    lax.fori_loop(0, (CPW + 1) // 2, pair, 0)
    wait_scatter(0)
    wait_scatter(1)
    plsc.subcore_barrier()
    pltpu.sync_copy(acc.at[pl.ds(s * RPT, RPT)], out.at[c, pl.ds(s * RPT, RPT)])


def kernel(x, edge_index, W1, a_src1, a_dst1, b1, W2, a_src2, a_dst2, b2):
    f32 = jnp.float32
    kk = jnp.arange(D)
    m1 = (kk[:, None] // HID == jnp.arange(HEADS)[None, :]).astype(f32)
    asad_w = jnp.concatenate([a_src1.reshape(-1)[:, None] * m1,
                              a_dst1.reshape(-1)[:, None] * m1], axis=1)
    r8 = (jnp.arange(HEADS)[:, None] == (jnp.arange(D)[None, :] // HID)
          ).astype(f32)
    a2 = jnp.concatenate([a_src2, a_dst2], axis=0).T

    BN = 2000
    grid = (N // BN,)

    h1, asad1 = pl.pallas_call(
        _proj1_body,
        grid=grid,
        in_specs=[
            pl.BlockSpec((BN, D), lambda i: (i, 0)),
            pl.BlockSpec((D, D), lambda i: (0, 0)),
            pl.BlockSpec((D, 2 * HEADS), lambda i: (0, 0)),
        ],
        out_specs=[
            pl.BlockSpec((BN, D), lambda i: (i, 0)),
            pl.BlockSpec((BN, 2 * HEADS), lambda i: (i, 0)),
        ],
        out_shape=[
            jax.ShapeDtypeStruct((N, D), f32),
            jax.ShapeDtypeStruct((N, 2 * HEADS), f32),
        ],
    )(x, W1, asad_w)

    mesh = plsc.VectorSubcoreMesh(**_MESH)

    edge1 = functools.partial(
        pl.kernel,
        out_type=(jax.ShapeDtypeStruct((NC, NP, D), f32),
                  jax.ShapeDtypeStruct((NC, NP, 16), f32)),
        mesh=mesh,
        compiler_params=pltpu.CompilerParams(use_tc_tiling_on_sc=False, needs_layout_passes=False),
        scratch_types=[
            pltpu.VMEM((C,), jnp.int32),
            pltpu.VMEM((2, 1, C), jnp.int32),
            pltpu.VMEM((C, 2 * HEADS), f32),
            pltpu.VMEM((C, 2 * HEADS), f32),
            pltpu.VMEM((2, C, D), f32),
            pltpu.VMEM((C, 16), f32),
            pltpu.VMEM_SHARED((NP, D), f32),
            pltpu.VMEM_SHARED((NP, 16), f32),
            pltpu.SemaphoreType.DMA,
            pltpu.SemaphoreType.DMA,
            pltpu.SemaphoreType.DMA,
            pltpu.SemaphoreType.DMA,
        ],
    )(_edge1_kernel)
    accm1, accd1 = edge1(edge_index, asad1, h1)

    h2, asad2 = pl.pallas_call(
        _combine1_body,
        grid=grid,
        in_specs=[
            pl.BlockSpec((NC, BN, D), lambda i: (0, i, 0)),
            pl.BlockSpec((NC, BN, 16), lambda i: (0, i, 0)),
            pl.BlockSpec((HEADS, D), lambda i: (0, 0)),
            pl.BlockSpec((1, D), lambda i: (0, 0)),
            pl.BlockSpec((D, OUT), lambda i: (0, 0)),
            pl.BlockSpec((OUT, 2), lambda i: (0, 0)),
        ],
        out_specs=[
            pl.BlockSpec((BN, OUT), lambda i: (i, 0)),
            pl.BlockSpec((BN, 2), lambda i: (i, 0)),
        ],
        out_shape=[
            jax.ShapeDtypeStruct((N, OUT), f32),
            jax.ShapeDtypeStruct((N, 2), f32),
        ],
    )(accm1, accd1, r8, b1.reshape(1, D), W2, a2)

    edge2 = functools.partial(
        pl.kernel,
        out_type=jax.ShapeDtypeStruct((NC, NP, ACC2), f32),
        mesh=mesh,
        compiler_params=pltpu.CompilerParams(use_tc_tiling_on_sc=False, needs_layout_passes=False),
        scratch_types=[
            pltpu.VMEM((2, C), jnp.int32),
            pltpu.VMEM((2, 1, C), jnp.int32),
            pltpu.VMEM((2, 1, C), jnp.int32),
            pltpu.VMEM((2, N), f32),
            pltpu.VMEM((2, C, OUT), f32),
            pltpu.VMEM((2, C, ACC2), f32),
            pltpu.VMEM_SHARED((NP, ACC2), f32),
            pltpu.SemaphoreType.DMA,
            pltpu.SemaphoreType.DMA,
            pltpu.SemaphoreType.DMA,
            pltpu.SemaphoreType.DMA,
        ],
    )(_edge2_kernel)
    acc2 = edge2(edge_index, asad2.T.reshape(2, N), h2)

    out = pl.pallas_call(
        _combine2_body,
        grid=grid,
        in_specs=[
            pl.BlockSpec((NC, BN, ACC2), lambda i: (0, i, 0)),
            pl.BlockSpec((1, OUT), lambda i: (0, 0)),
        ],
        out_specs=pl.BlockSpec((BN, OUT), lambda i: (i, 0)),
        out_shape=jax.ShapeDtypeStruct((N, OUT), f32),
    )(acc2, b2.reshape(1, OUT))
    return out
